# baseline (speedup 1.0000x reference)
import functools

import jax
import jax.numpy as jnp
import numpy as np
from jax import lax
from jax.experimental import pallas as pl
from jax.experimental.pallas import tpu as pltpu
from jax.experimental.pallas import tpu_sc as plsc

_BLK = 256
_CHUNK = 32
_CAP = 2048


def _dft_pair(xr, xi, n1, n2, sign, scale):
    r, nn = xr.shape
    f32 = jnp.float32
    prec = lax.Precision.DEFAULT
    j2 = np.arange(n2)
    ang2 = sign * 2 * np.pi / n2 * (j2[:, None] * j2[None, :])
    f2r, f2i = jnp.asarray(np.cos(ang2), f32), jnp.asarray(np.sin(ang2), f32)
    j1 = np.arange(n1)
    ang1 = sign * 2 * np.pi / n1 * (j1[:, None] * j1[None, :])
    f1r, f1i = jnp.asarray(np.cos(ang1), f32), jnp.asarray(np.sin(ang1), f32)
    angt = sign * 2 * np.pi / nn * (j1[:, None] * j2[None, :])
    tr, ti = jnp.asarray(np.cos(angt), f32), jnp.asarray(np.sin(angt), f32)
    yr = xr.reshape(r, n2, n1).transpose(0, 2, 1)
    yi = xi.reshape(r, n2, n1).transpose(0, 2, 1)
    p1 = jnp.matmul(yr, f2r, precision=prec)
    p2 = jnp.matmul(yi, f2i, precision=prec)
    p3 = jnp.matmul(yr + yi, f2r + f2i, precision=prec)
    ar = p1 - p2
    ai = p3 - p1 - p2
    br = ar * tr - ai * ti
    bi = ar * ti + ai * tr
    ein = functools.partial(jnp.einsum, 'rjk,jl->rlk', precision=prec)
    q1 = ein(br, f1r)
    q2 = ein(bi, f1i)
    q3 = ein(br + bi, f1r + f1i)
    xr2 = q1 - q2
    xi2 = q3 - q1 - q2
    return xr2.reshape(r, nn) * scale, xi2.reshape(r, nn) * scale


def _gelu(v):
    return 0.5 * v * (1.0 + jax.lax.erf(v * 0.7071067811865476))


def _relu(v):
    return jnp.maximum(v, 0.0)


def _mlp(xb, w1t, b1, w2t, b2, w3t, b3, act):
    h = act(jnp.dot(xb, w1t, preferred_element_type=jnp.float32) + b1)
    h = act(jnp.dot(h, w2t, preferred_element_type=jnp.float32) + b2)
    return jnp.dot(h, w3t, preferred_element_type=jnp.float32) + b3


def _mlp_select(xb, m, weights):
    (fw1, fb1, fw2, fb2, fw3, fb3,
     pw1, pb1, pw2, pb2, pw3, pb3,
     aw1, ab1, aw2, ab2, aw3, ab3,
     bw1, bb1, bw2, bb2, bw3, bb3) = weights
    acc = _mlp(xb, fw1, fb1, fw2, fb2, fw3, fb3, _gelu)
    o2 = _mlp(xb, pw1, pb1, pw2, pb2, pw3, pb3, _gelu)
    acc = jnp.where(m == 2, o2, acc)
    o3 = _mlp(xb, aw1, ab1, aw2, ab2, aw3, ab3, _relu)
    acc = jnp.where(m == 3, o3, acc)
    o4 = _mlp(xb, bw1, bb1, bw2, bb2, bw3, bb3, _relu)
    return jnp.where(m == 4, o4, acc)


def _tc_mlp_body(mask_ref, x_ref, *rest):
    out_ref = rest[-1]
    weights = tuple(r[...] for r in rest[:-1])
    out_ref[...] = _mlp_select(x_ref[...], mask_ref[...], weights)


def _tc_mono_body(mask_ref, x_ref, mt_ref, *rest):
    out_ref = rest[-1]
    weights = tuple(r[...] for r in rest[:-1])
    acc = _mlp_select(x_ref[...], mask_ref[...], weights)
    o0 = jnp.dot(x_ref[...], mt_ref[...], preferred_element_type=jnp.float32)
    out_ref[...] = jnp.where(mask_ref[...] == 0, o0, acc)


def _tc_matmul_body(xg_ref, mt_ref, og_ref):
    og_ref[...] = jnp.dot(xg_ref[...], mt_ref[...],
                          preferred_element_type=jnp.float32)


def kernel(x, region_mask, kr, ki, fw1, fb1, fw2, fb2, fw3, fb3, pw1, pb1,
           pw2, pb2, pw3, pb3, aw1, ab1, aw2, ab2, aw3, ab3, bw1, bb1, bw2,
           bb2, bw3, bb3):
    n, d = x.shape
    blk = _BLK
    nblocks = n // blk
    cap = _CAP
    capb = cap // blk

    n1 = 32
    n2 = d // n1
    bfr, bfi = _dft_pair(kr, ki, n1, n2, -1.0, 1.0)
    mt, _ = _dft_pair(bfr.T, bfi.T, n1, n2, 1.0, 1.0 / d)

    is0 = region_mask == 0
    iota_n = jnp.arange(n, dtype=jnp.int32)
    pos = jnp.cumsum(is0.astype(jnp.int32)) - 1
    n0 = pos[-1] + 1
    last0 = jnp.maximum(jnp.max(jnp.where(is0, iota_n, -1)), 0)
    scat_tgt = jnp.where(is0, pos, cap)
    idx0 = jnp.zeros((cap,), jnp.int32).at[scat_tgt].set(iota_n, mode="drop")
    idx = jnp.where(jnp.arange(cap, dtype=jnp.int32) < n0, idx0, last0)

    mask2 = region_mask.reshape(n, 1)
    weights = (fw1.T, fb1.reshape(1, -1), fw2.T, fb2.reshape(1, -1), fw3.T, fb3.reshape(1, -1),
               pw1.T, pb1.reshape(1, -1), pw2.T, pb2.reshape(1, -1), pw3.T, pb3.reshape(1, -1),
               aw1.T, ab1.reshape(1, -1), aw2.T, ab2.reshape(1, -1), aw3.T, ab3.reshape(1, -1),
               bw1.T, bb1.reshape(1, -1), bw2.T, bb2.reshape(1, -1), bw3.T, bb3.reshape(1, -1))
    full = lambda a: pl.BlockSpec(a.shape, lambda i: (0,) * a.ndim)
    w_specs = [full(w) for w in weights]

    info = plsc.get_sparse_core_info()
    nw = info.num_cores * info.num_subcores
    chunk = _CHUNK
    per_worker = cap // chunk // nw
    assert per_worker * chunk * nw == cap

    mesh = plsc.VectorSubcoreMesh(core_axis_name="c", subcore_axis_name="s")
    sc_scratch = [
        pltpu.VMEM((chunk,), jnp.int32),
        pltpu.VMEM((chunk, d), jnp.float32),
        pltpu.SemaphoreType.DMA,
    ]

    @functools.partial(
        pl.kernel, mesh=mesh,
        out_type=jax.ShapeDtypeStruct((cap, d), jnp.float32),
        scratch_types=sc_scratch,
    )
    def sc_gather(x_hbm, idx_hbm, xg_hbm, idxv, rowsv, sem):
        wid = lax.axis_index("s") * info.num_cores + lax.axis_index("c")

        def body(t, carry):
            base = (t * nw + wid) * chunk
            pltpu.sync_copy(idx_hbm.at[pl.ds(base, chunk)], idxv)
            pltpu.async_copy(x_hbm.at[idxv], rowsv, sem).wait()
            pltpu.sync_copy(rowsv, xg_hbm.at[pl.ds(base, chunk)])
            return carry

        lax.fori_loop(0, per_worker, body, 0)

    @functools.partial(
        pl.kernel, mesh=mesh,
        out_type=(),
        scratch_types=sc_scratch,
    )
    def sc_scatter(og_hbm, idx_hbm, out_hbm, idxv, rowsv, sem):
        wid = lax.axis_index("s") * info.num_cores + lax.axis_index("c")

        def body(t, carry):
            base = (t * nw + wid) * chunk
            pltpu.sync_copy(idx_hbm.at[pl.ds(base, chunk)], idxv)
            pltpu.sync_copy(og_hbm.at[pl.ds(base, chunk)], rowsv)
            pltpu.async_copy(rowsv, out_hbm.at[idxv], sem).wait()
            return carry

        lax.fori_loop(0, per_worker, body, 0)

    def fast_path():
        xg = sc_gather(x, idx)
        og = pl.pallas_call(
            _tc_matmul_body,
            grid=(capb,),
            in_specs=[pl.BlockSpec((blk, d), lambda i: (i, 0)),
                      pl.BlockSpec((d, d), lambda i: (0, 0))],
            out_specs=pl.BlockSpec((blk, d), lambda i: (i, 0)),
            out_shape=jax.ShapeDtypeStruct((cap, d), jnp.float32),
        )(xg, mt)
        mlp_out = pl.pallas_call(
            _tc_mlp_body,
            grid=(nblocks,),
            in_specs=[pl.BlockSpec((blk, 1), lambda i: (i, 0)),
                      pl.BlockSpec((blk, d), lambda i: (i, 0))] + w_specs,
            out_specs=pl.BlockSpec((blk, d), lambda i: (i, 0)),
            out_shape=jax.ShapeDtypeStruct((n, d), jnp.float32),
        )(mask2, x, *weights)
        out_ref = jax.new_ref(mlp_out)
        sc_scatter(og, idx, out_ref)
        return jax.freeze(out_ref)

    def slow_path():
        return pl.pallas_call(
            _tc_mono_body,
            grid=(nblocks,),
            in_specs=[pl.BlockSpec((blk, 1), lambda i: (i, 0)),
                      pl.BlockSpec((blk, d), lambda i: (i, 0)),
                      pl.BlockSpec((d, d), lambda i: (0, 0))] + w_specs,
            out_specs=pl.BlockSpec((blk, d), lambda i: (i, 0)),
            out_shape=jax.ShapeDtypeStruct((n, d), jnp.float32),
        )(mask2, x, mt, *weights)

    return lax.cond((n0 > 0) & (n0 <= cap), fast_path, slow_path)


# speedup vs baseline: 1.0025x; 1.0025x over previous
import functools

import jax
import jax.numpy as jnp
import numpy as np
from jax import lax
from jax.experimental import pallas as pl
from jax.experimental.pallas import tpu as pltpu
from jax.experimental.pallas import tpu_sc as plsc

_BLK = 256
_CHUNK = 32
_CAP = 2048


def _dft_pair(xr, xi, n1, n2, sign, scale):
    r, nn = xr.shape
    f32 = jnp.float32
    prec = lax.Precision.DEFAULT
    j2 = np.arange(n2)
    ang2 = sign * 2 * np.pi / n2 * (j2[:, None] * j2[None, :])
    f2r, f2i = jnp.asarray(np.cos(ang2), f32), jnp.asarray(np.sin(ang2), f32)
    j1 = np.arange(n1)
    ang1 = sign * 2 * np.pi / n1 * (j1[:, None] * j1[None, :])
    f1r, f1i = jnp.asarray(np.cos(ang1), f32), jnp.asarray(np.sin(ang1), f32)
    angt = sign * 2 * np.pi / nn * (j1[:, None] * j2[None, :])
    tr, ti = jnp.asarray(np.cos(angt), f32), jnp.asarray(np.sin(angt), f32)
    yr = xr.reshape(r, n2, n1).transpose(0, 2, 1)
    yi = xi.reshape(r, n2, n1).transpose(0, 2, 1)
    ar = jnp.matmul(yr, f2r, precision=prec) - jnp.matmul(yi, f2i, precision=prec)
    ai = jnp.matmul(yr, f2i, precision=prec) + jnp.matmul(yi, f2r, precision=prec)
    br = ar * tr - ai * ti
    bi = ar * ti + ai * tr
    ein = functools.partial(jnp.einsum, 'rjk,jl->rlk', precision=prec)
    xr2 = ein(br, f1r) - ein(bi, f1i)
    xi2 = ein(br, f1i) + ein(bi, f1r)
    return xr2.reshape(r, nn) * scale, xi2.reshape(r, nn) * scale


def _gelu(v):
    return 0.5 * v * (1.0 + jax.lax.erf(v * 0.7071067811865476))


def _relu(v):
    return jnp.maximum(v, 0.0)


def _mlp(xb, w1t, b1, w2t, b2, w3t, b3, act):
    h = act(jnp.dot(xb, w1t, preferred_element_type=jnp.float32) + b1)
    h = act(jnp.dot(h, w2t, preferred_element_type=jnp.float32) + b2)
    return jnp.dot(h, w3t, preferred_element_type=jnp.float32) + b3


def _mlp_select(xb, m, weights):
    (fw1, fb1, fw2, fb2, fw3, fb3,
     pw1, pb1, pw2, pb2, pw3, pb3,
     aw1, ab1, aw2, ab2, aw3, ab3,
     bw1, bb1, bw2, bb2, bw3, bb3) = weights
    acc = _mlp(xb, fw1, fb1, fw2, fb2, fw3, fb3, _gelu)
    o2 = _mlp(xb, pw1, pb1, pw2, pb2, pw3, pb3, _gelu)
    acc = jnp.where(m == 2, o2, acc)
    o3 = _mlp(xb, aw1, ab1, aw2, ab2, aw3, ab3, _relu)
    acc = jnp.where(m == 3, o3, acc)
    o4 = _mlp(xb, bw1, bb1, bw2, bb2, bw3, bb3, _relu)
    return jnp.where(m == 4, o4, acc)


def _tc_mlp_body(mask_ref, x_ref, *rest):
    out_ref = rest[-1]
    weights = tuple(r[...] for r in rest[:-1])
    out_ref[...] = _mlp_select(x_ref[...], mask_ref[...], weights)


def _tc_mono_body(mask_ref, x_ref, mt_ref, *rest):
    out_ref = rest[-1]
    weights = tuple(r[...] for r in rest[:-1])
    acc = _mlp_select(x_ref[...], mask_ref[...], weights)
    o0 = jnp.dot(x_ref[...], mt_ref[...], preferred_element_type=jnp.float32)
    out_ref[...] = jnp.where(mask_ref[...] == 0, o0, acc)


def _tc_matmul_body(xg_ref, mt_ref, og_ref):
    og_ref[...] = jnp.dot(xg_ref[...], mt_ref[...],
                          preferred_element_type=jnp.float32)


def kernel(x, region_mask, kr, ki, fw1, fb1, fw2, fb2, fw3, fb3, pw1, pb1,
           pw2, pb2, pw3, pb3, aw1, ab1, aw2, ab2, aw3, ab3, bw1, bb1, bw2,
           bb2, bw3, bb3):
    n, d = x.shape
    blk = _BLK
    nblocks = n // blk
    cap = _CAP
    capb = cap // blk

    n1 = 32
    n2 = d // n1
    bfr, bfi = _dft_pair(kr, ki, n1, n2, -1.0, 1.0)
    mt, _ = _dft_pair(bfr.T, bfi.T, n1, n2, 1.0, 1.0 / d)

    is0 = region_mask == 0
    iota_n = jnp.arange(n, dtype=jnp.int32)
    pos = jnp.cumsum(is0.astype(jnp.int32)) - 1
    n0 = pos[-1] + 1
    last0 = jnp.maximum(jnp.max(jnp.where(is0, iota_n, -1)), 0)
    scat_tgt = jnp.where(is0, pos, cap)
    idx0 = jnp.zeros((cap,), jnp.int32).at[scat_tgt].set(iota_n, mode="drop")
    idx = jnp.where(jnp.arange(cap, dtype=jnp.int32) < n0, idx0, last0)

    mask2 = region_mask.reshape(n, 1)
    weights = (fw1.T, fb1.reshape(1, -1), fw2.T, fb2.reshape(1, -1), fw3.T, fb3.reshape(1, -1),
               pw1.T, pb1.reshape(1, -1), pw2.T, pb2.reshape(1, -1), pw3.T, pb3.reshape(1, -1),
               aw1.T, ab1.reshape(1, -1), aw2.T, ab2.reshape(1, -1), aw3.T, ab3.reshape(1, -1),
               bw1.T, bb1.reshape(1, -1), bw2.T, bb2.reshape(1, -1), bw3.T, bb3.reshape(1, -1))
    full = lambda a: pl.BlockSpec(a.shape, lambda i: (0,) * a.ndim)
    w_specs = [full(w) for w in weights]

    info = plsc.get_sparse_core_info()
    nw = info.num_cores * info.num_subcores
    chunk = _CHUNK
    per_worker = cap // chunk // nw
    assert per_worker * chunk * nw == cap

    mesh = plsc.VectorSubcoreMesh(core_axis_name="c", subcore_axis_name="s")
    sc_scratch = [
        pltpu.VMEM((chunk,), jnp.int32),
        pltpu.VMEM((chunk, d), jnp.float32),
        pltpu.SemaphoreType.DMA,
    ]

    @functools.partial(
        pl.kernel, mesh=mesh,
        out_type=jax.ShapeDtypeStruct((cap, d), jnp.float32),
        scratch_types=sc_scratch,
    )
    def sc_gather(x_hbm, idx_hbm, xg_hbm, idxv, rowsv, sem):
        wid = lax.axis_index("s") * info.num_cores + lax.axis_index("c")

        def body(t, carry):
            base = (t * nw + wid) * chunk
            pltpu.sync_copy(idx_hbm.at[pl.ds(base, chunk)], idxv)
            pltpu.async_copy(x_hbm.at[idxv], rowsv, sem).wait()
            pltpu.sync_copy(rowsv, xg_hbm.at[pl.ds(base, chunk)])
            return carry

        lax.fori_loop(0, per_worker, body, 0)

    @functools.partial(
        pl.kernel, mesh=mesh,
        out_type=(),
        scratch_types=sc_scratch,
    )
    def sc_scatter(og_hbm, idx_hbm, out_hbm, idxv, rowsv, sem):
        wid = lax.axis_index("s") * info.num_cores + lax.axis_index("c")

        def body(t, carry):
            base = (t * nw + wid) * chunk
            pltpu.sync_copy(idx_hbm.at[pl.ds(base, chunk)], idxv)
            pltpu.sync_copy(og_hbm.at[pl.ds(base, chunk)], rowsv)
            pltpu.async_copy(rowsv, out_hbm.at[idxv], sem).wait()
            return carry

        lax.fori_loop(0, per_worker, body, 0)

    def fast_path():
        xg = sc_gather(x, idx)
        og = pl.pallas_call(
            _tc_matmul_body,
            grid=(capb,),
            in_specs=[pl.BlockSpec((blk, d), lambda i: (i, 0)),
                      pl.BlockSpec((d, d), lambda i: (0, 0))],
            out_specs=pl.BlockSpec((blk, d), lambda i: (i, 0)),
            out_shape=jax.ShapeDtypeStruct((cap, d), jnp.float32),
        )(xg, mt)
        mlp_out = pl.pallas_call(
            _tc_mlp_body,
            grid=(nblocks,),
            in_specs=[pl.BlockSpec((blk, 1), lambda i: (i, 0)),
                      pl.BlockSpec((blk, d), lambda i: (i, 0))] + w_specs,
            out_specs=pl.BlockSpec((blk, d), lambda i: (i, 0)),
            out_shape=jax.ShapeDtypeStruct((n, d), jnp.float32),
        )(mask2, x, *weights)
        out_ref = jax.new_ref(mlp_out)
        sc_scatter(og, idx, out_ref)
        return jax.freeze(out_ref)

    def slow_path():
        return pl.pallas_call(
            _tc_mono_body,
            grid=(nblocks,),
            in_specs=[pl.BlockSpec((blk, 1), lambda i: (i, 0)),
                      pl.BlockSpec((blk, d), lambda i: (i, 0)),
                      pl.BlockSpec((d, d), lambda i: (0, 0))] + w_specs,
            out_specs=pl.BlockSpec((blk, d), lambda i: (i, 0)),
            out_shape=jax.ShapeDtypeStruct((n, d), jnp.float32),
        )(mask2, x, mt, *weights)

    return lax.cond((n0 > 0) & (n0 <= cap), fast_path, slow_path)


# speedup vs baseline: 1.0641x; 1.0614x over previous
import functools

import jax
import jax.numpy as jnp
import numpy as np
from jax import lax
from jax.experimental import pallas as pl
from jax.experimental.pallas import tpu as pltpu
from jax.experimental.pallas import tpu_sc as plsc

_BLK = 256
_CHUNK = 32
_CAP = 2048


def _dft_pair(xr, xi, n1, n2, sign, scale):
    r, nn = xr.shape
    f32 = jnp.float32
    prec = lax.Precision.DEFAULT
    j2 = np.arange(n2)
    ang2 = sign * 2 * np.pi / n2 * (j2[:, None] * j2[None, :])
    f2r, f2i = jnp.asarray(np.cos(ang2), f32), jnp.asarray(np.sin(ang2), f32)
    j1 = np.arange(n1)
    ang1 = sign * 2 * np.pi / n1 * (j1[:, None] * j1[None, :])
    f1r, f1i = jnp.asarray(np.cos(ang1), f32), jnp.asarray(np.sin(ang1), f32)
    angt = sign * 2 * np.pi / nn * (j1[:, None] * j2[None, :])
    tr, ti = jnp.asarray(np.cos(angt), f32), jnp.asarray(np.sin(angt), f32)
    yr = xr.reshape(r, n2, n1).transpose(0, 2, 1)
    yi = xi.reshape(r, n2, n1).transpose(0, 2, 1)
    ar = jnp.matmul(yr, f2r, precision=prec) - jnp.matmul(yi, f2i, precision=prec)
    ai = jnp.matmul(yr, f2i, precision=prec) + jnp.matmul(yi, f2r, precision=prec)
    br = ar * tr - ai * ti
    bi = ar * ti + ai * tr
    ein = functools.partial(jnp.einsum, 'rjk,jl->rlk', precision=prec)
    xr2 = ein(br, f1r) - ein(bi, f1i)
    xi2 = ein(br, f1i) + ein(bi, f1r)
    return xr2.reshape(r, nn) * scale, xi2.reshape(r, nn) * scale


def _gelu(v):
    return 0.5 * v * (1.0 + jax.lax.erf(v * 0.7071067811865476))


def _relu(v):
    return jnp.maximum(v, 0.0)


def _mlp(xb, w1t, b1, w2t, b2, w3t, b3, act):
    h = act(jnp.dot(xb, w1t, preferred_element_type=jnp.float32) + b1)
    h = act(jnp.dot(h, w2t, preferred_element_type=jnp.float32) + b2)
    return jnp.dot(h, w3t, preferred_element_type=jnp.float32) + b3


def _mlp_select(xb, m, weights):
    (fw1, fb1, fw2, fb2, fw3, fb3,
     pw1, pb1, pw2, pb2, pw3, pb3,
     aw1, ab1, aw2, ab2, aw3, ab3,
     bw1, bb1, bw2, bb2, bw3, bb3) = weights
    acc = _mlp(xb, fw1, fb1, fw2, fb2, fw3, fb3, _gelu)
    o2 = _mlp(xb, pw1, pb1, pw2, pb2, pw3, pb3, _gelu)
    acc = jnp.where(m == 2, o2, acc)
    o3 = _mlp(xb, aw1, ab1, aw2, ab2, aw3, ab3, _relu)
    acc = jnp.where(m == 3, o3, acc)
    o4 = _mlp(xb, bw1, bb1, bw2, bb2, bw3, bb3, _relu)
    return jnp.where(m == 4, o4, acc)


def _tc_mlp_body(mask_ref, x_ref, *rest):
    out_ref = rest[-1]
    weights = tuple(r[...] for r in rest[:-1])
    out_ref[...] = _mlp_select(x_ref[...], mask_ref[...], weights)


def _tc_mono_body(mask_ref, x_ref, mt_ref, *rest):
    out_ref = rest[-1]
    weights = tuple(r[...] for r in rest[:-1])
    acc = _mlp_select(x_ref[...], mask_ref[...], weights)
    o0 = jnp.dot(x_ref[...], mt_ref[...], preferred_element_type=jnp.float32)
    out_ref[...] = jnp.where(mask_ref[...] == 0, o0, acc)


def _tc_matmul_body(xg_ref, mt_ref, og_ref):
    og_ref[...] = jnp.dot(xg_ref[...], mt_ref[...],
                          preferred_element_type=jnp.float32)


def kernel(x, region_mask, kr, ki, fw1, fb1, fw2, fb2, fw3, fb3, pw1, pb1,
           pw2, pb2, pw3, pb3, aw1, ab1, aw2, ab2, aw3, ab3, bw1, bb1, bw2,
           bb2, bw3, bb3):
    n, d = x.shape
    blk = _BLK
    nblocks = n // blk
    cap = _CAP
    capb = cap // blk

    n1 = 64
    n2 = d // n1
    bfr, bfi = _dft_pair(kr, ki, n1, n2, -1.0, 1.0)
    mt, _ = _dft_pair(bfr.T, bfi.T, n1, n2, 1.0, 1.0 / d)

    is0 = region_mask == 0
    iota_n = jnp.arange(n, dtype=jnp.int32)
    pos = jnp.cumsum(is0.astype(jnp.int32)) - 1
    n0 = pos[-1] + 1
    last0 = jnp.maximum(jnp.max(jnp.where(is0, iota_n, -1)), 0)
    scat_tgt = jnp.where(is0, pos, cap)
    idx0 = jnp.zeros((cap,), jnp.int32).at[scat_tgt].set(iota_n, mode="drop")
    idx = jnp.where(jnp.arange(cap, dtype=jnp.int32) < n0, idx0, last0)

    mask2 = region_mask.reshape(n, 1)
    weights = (fw1.T, fb1.reshape(1, -1), fw2.T, fb2.reshape(1, -1), fw3.T, fb3.reshape(1, -1),
               pw1.T, pb1.reshape(1, -1), pw2.T, pb2.reshape(1, -1), pw3.T, pb3.reshape(1, -1),
               aw1.T, ab1.reshape(1, -1), aw2.T, ab2.reshape(1, -1), aw3.T, ab3.reshape(1, -1),
               bw1.T, bb1.reshape(1, -1), bw2.T, bb2.reshape(1, -1), bw3.T, bb3.reshape(1, -1))
    full = lambda a: pl.BlockSpec(a.shape, lambda i: (0,) * a.ndim)
    w_specs = [full(w) for w in weights]

    info = plsc.get_sparse_core_info()
    nw = info.num_cores * info.num_subcores
    chunk = _CHUNK
    per_worker = cap // chunk // nw
    assert per_worker * chunk * nw == cap

    mesh = plsc.VectorSubcoreMesh(core_axis_name="c", subcore_axis_name="s")
    sc_scratch = [
        pltpu.VMEM((chunk,), jnp.int32),
        pltpu.VMEM((chunk, d), jnp.float32),
        pltpu.SemaphoreType.DMA,
    ]

    @functools.partial(
        pl.kernel, mesh=mesh,
        out_type=jax.ShapeDtypeStruct((cap, d), jnp.float32),
        scratch_types=sc_scratch,
    )
    def sc_gather(x_hbm, idx_hbm, xg_hbm, idxv, rowsv, sem):
        wid = lax.axis_index("s") * info.num_cores + lax.axis_index("c")

        def body(t, carry):
            base = (t * nw + wid) * chunk
            pltpu.sync_copy(idx_hbm.at[pl.ds(base, chunk)], idxv)
            pltpu.async_copy(x_hbm.at[idxv], rowsv, sem).wait()
            pltpu.sync_copy(rowsv, xg_hbm.at[pl.ds(base, chunk)])
            return carry

        lax.fori_loop(0, per_worker, body, 0)

    @functools.partial(
        pl.kernel, mesh=mesh,
        out_type=(),
        scratch_types=sc_scratch,
    )
    def sc_scatter(og_hbm, idx_hbm, out_hbm, idxv, rowsv, sem):
        wid = lax.axis_index("s") * info.num_cores + lax.axis_index("c")

        def body(t, carry):
            base = (t * nw + wid) * chunk
            pltpu.sync_copy(idx_hbm.at[pl.ds(base, chunk)], idxv)
            pltpu.sync_copy(og_hbm.at[pl.ds(base, chunk)], rowsv)
            pltpu.async_copy(rowsv, out_hbm.at[idxv], sem).wait()
            return carry

        lax.fori_loop(0, per_worker, body, 0)

    def fast_path():
        xg = sc_gather(x, idx)
        og = pl.pallas_call(
            _tc_matmul_body,
            grid=(capb,),
            in_specs=[pl.BlockSpec((blk, d), lambda i: (i, 0)),
                      pl.BlockSpec((d, d), lambda i: (0, 0))],
            out_specs=pl.BlockSpec((blk, d), lambda i: (i, 0)),
            out_shape=jax.ShapeDtypeStruct((cap, d), jnp.float32),
        )(xg, mt)
        mlp_out = pl.pallas_call(
            _tc_mlp_body,
            grid=(nblocks,),
            in_specs=[pl.BlockSpec((blk, 1), lambda i: (i, 0)),
                      pl.BlockSpec((blk, d), lambda i: (i, 0))] + w_specs,
            out_specs=pl.BlockSpec((blk, d), lambda i: (i, 0)),
            out_shape=jax.ShapeDtypeStruct((n, d), jnp.float32),
        )(mask2, x, *weights)
        out_ref = jax.new_ref(mlp_out)
        sc_scatter(og, idx, out_ref)
        return jax.freeze(out_ref)

    def slow_path():
        return pl.pallas_call(
            _tc_mono_body,
            grid=(nblocks,),
            in_specs=[pl.BlockSpec((blk, 1), lambda i: (i, 0)),
                      pl.BlockSpec((blk, d), lambda i: (i, 0)),
                      pl.BlockSpec((d, d), lambda i: (0, 0))] + w_specs,
            out_specs=pl.BlockSpec((blk, d), lambda i: (i, 0)),
            out_shape=jax.ShapeDtypeStruct((n, d), jnp.float32),
        )(mask2, x, mt, *weights)

    return lax.cond((n0 > 0) & (n0 <= cap), fast_path, slow_path)


# speedup vs baseline: 1.1865x; 1.1151x over previous
import functools

import jax
import jax.numpy as jnp
import numpy as np
from jax import lax
from jax.experimental import pallas as pl
from jax.experimental.pallas import tpu as pltpu
from jax.experimental.pallas import tpu_sc as plsc

_BLK = 256
_CHUNK = 32
_CAP = 2048


def _dft_pair(xr, xi, n1, n2, sign, scale):
    r, nn = xr.shape
    f32 = jnp.float32
    prec = lax.Precision.DEFAULT
    j2 = np.arange(n2)
    ang2 = sign * 2 * np.pi / n2 * (j2[:, None] * j2[None, :])
    f2r, f2i = jnp.asarray(np.cos(ang2), f32), jnp.asarray(np.sin(ang2), f32)
    j1 = np.arange(n1)
    ang1 = sign * 2 * np.pi / n1 * (j1[:, None] * j1[None, :])
    f1r, f1i = jnp.asarray(np.cos(ang1), f32), jnp.asarray(np.sin(ang1), f32)
    angt = sign * 2 * np.pi / nn * (j1[:, None] * j2[None, :])
    tr, ti = jnp.asarray(np.cos(angt), f32), jnp.asarray(np.sin(angt), f32)
    yr = xr.reshape(r, n2, n1).transpose(0, 2, 1)
    yi = xi.reshape(r, n2, n1).transpose(0, 2, 1)
    ar = jnp.matmul(yr, f2r, precision=prec) - jnp.matmul(yi, f2i, precision=prec)
    ai = jnp.matmul(yr, f2i, precision=prec) + jnp.matmul(yi, f2r, precision=prec)
    br = ar * tr - ai * ti
    bi = ar * ti + ai * tr
    ein = functools.partial(jnp.einsum, 'rjk,jl->rlk', precision=prec)
    xr2 = ein(br, f1r) - ein(bi, f1i)
    xi2 = ein(br, f1i) + ein(bi, f1r)
    return xr2.reshape(r, nn) * scale, xi2.reshape(r, nn) * scale


def _gelu(v):
    return 0.5 * v * (1.0 + jax.lax.erf(v * 0.7071067811865476))


def _relu(v):
    return jnp.maximum(v, 0.0)


def _mlp(xb, w1t, b1, w2t, b2, w3t, b3, act):
    h = act(jnp.dot(xb, w1t, preferred_element_type=jnp.float32) + b1)
    h = act(jnp.dot(h, w2t, preferred_element_type=jnp.float32) + b2)
    return jnp.dot(h, w3t, preferred_element_type=jnp.float32) + b3


def _mlp_select(xb, m, weights):
    (fw1, fb1, fw2, fb2, fw3, fb3,
     pw1, pb1, pw2, pb2, pw3, pb3,
     aw1, ab1, aw2, ab2, aw3, ab3,
     bw1, bb1, bw2, bb2, bw3, bb3) = weights
    acc = _mlp(xb, fw1, fb1, fw2, fb2, fw3, fb3, _gelu)
    o2 = _mlp(xb, pw1, pb1, pw2, pb2, pw3, pb3, _gelu)
    acc = jnp.where(m == 2, o2, acc)
    o3 = _mlp(xb, aw1, ab1, aw2, ab2, aw3, ab3, _relu)
    acc = jnp.where(m == 3, o3, acc)
    o4 = _mlp(xb, bw1, bb1, bw2, bb2, bw3, bb3, _relu)
    return jnp.where(m == 4, o4, acc)


def _tc_mlp_body(mask_ref, x_ref, *rest):
    out_ref = rest[-1]
    weights = tuple(r[...] for r in rest[:-1])
    out_ref[...] = _mlp_select(x_ref[...], mask_ref[...], weights)


def _tc_mono_body(mask_ref, x_ref, mt_ref, *rest):
    out_ref = rest[-1]
    weights = tuple(r[...] for r in rest[:-1])
    acc = _mlp_select(x_ref[...], mask_ref[...], weights)
    o0 = jnp.dot(x_ref[...], mt_ref[...], preferred_element_type=jnp.float32)
    out_ref[...] = jnp.where(mask_ref[...] == 0, o0, acc)


def _tc_matmul_body(xg_ref, mt_ref, og_ref):
    og_ref[...] = jnp.dot(xg_ref[...], mt_ref[...],
                          preferred_element_type=jnp.float32)


def kernel(x, region_mask, kr, ki, fw1, fb1, fw2, fb2, fw3, fb3, pw1, pb1,
           pw2, pb2, pw3, pb3, aw1, ab1, aw2, ab2, aw3, ab3, bw1, bb1, bw2,
           bb2, bw3, bb3):
    n, d = x.shape
    blk = _BLK
    nblocks = n // blk
    cap = _CAP
    capb = cap // blk

    n1 = 16
    n2 = d // n1
    bfr, bfi = _dft_pair(kr, ki, n1, n2, -1.0, 1.0)
    mt, _ = _dft_pair(bfr.T, bfi.T, n1, n2, 1.0, 1.0 / d)

    is0 = region_mask == 0
    iota_n = jnp.arange(n, dtype=jnp.int32)
    pos = jnp.cumsum(is0.astype(jnp.int32)) - 1
    n0 = pos[-1] + 1
    last0 = jnp.maximum(jnp.max(jnp.where(is0, iota_n, -1)), 0)
    scat_tgt = jnp.where(is0, pos, cap)
    idx0 = jnp.zeros((cap,), jnp.int32).at[scat_tgt].set(iota_n, mode="drop")
    idx = jnp.where(jnp.arange(cap, dtype=jnp.int32) < n0, idx0, last0)

    mask2 = region_mask.reshape(n, 1)
    weights = (fw1.T, fb1.reshape(1, -1), fw2.T, fb2.reshape(1, -1), fw3.T, fb3.reshape(1, -1),
               pw1.T, pb1.reshape(1, -1), pw2.T, pb2.reshape(1, -1), pw3.T, pb3.reshape(1, -1),
               aw1.T, ab1.reshape(1, -1), aw2.T, ab2.reshape(1, -1), aw3.T, ab3.reshape(1, -1),
               bw1.T, bb1.reshape(1, -1), bw2.T, bb2.reshape(1, -1), bw3.T, bb3.reshape(1, -1))
    full = lambda a: pl.BlockSpec(a.shape, lambda i: (0,) * a.ndim)
    w_specs = [full(w) for w in weights]

    info = plsc.get_sparse_core_info()
    nw = info.num_cores * info.num_subcores
    chunk = _CHUNK
    per_worker = cap // chunk // nw
    assert per_worker * chunk * nw == cap

    mesh = plsc.VectorSubcoreMesh(core_axis_name="c", subcore_axis_name="s")
    sc_scratch = [
        pltpu.VMEM((chunk,), jnp.int32),
        pltpu.VMEM((chunk, d), jnp.float32),
        pltpu.SemaphoreType.DMA,
    ]

    @functools.partial(
        pl.kernel, mesh=mesh,
        out_type=jax.ShapeDtypeStruct((cap, d), jnp.float32),
        scratch_types=sc_scratch,
    )
    def sc_gather(x_hbm, idx_hbm, xg_hbm, idxv, rowsv, sem):
        wid = lax.axis_index("s") * info.num_cores + lax.axis_index("c")

        def body(t, carry):
            base = (t * nw + wid) * chunk
            pltpu.sync_copy(idx_hbm.at[pl.ds(base, chunk)], idxv)
            pltpu.async_copy(x_hbm.at[idxv], rowsv, sem).wait()
            pltpu.sync_copy(rowsv, xg_hbm.at[pl.ds(base, chunk)])
            return carry

        lax.fori_loop(0, per_worker, body, 0)

    @functools.partial(
        pl.kernel, mesh=mesh,
        out_type=(),
        scratch_types=sc_scratch,
    )
    def sc_scatter(og_hbm, idx_hbm, out_hbm, idxv, rowsv, sem):
        wid = lax.axis_index("s") * info.num_cores + lax.axis_index("c")

        def body(t, carry):
            base = (t * nw + wid) * chunk
            pltpu.sync_copy(idx_hbm.at[pl.ds(base, chunk)], idxv)
            pltpu.sync_copy(og_hbm.at[pl.ds(base, chunk)], rowsv)
            pltpu.async_copy(rowsv, out_hbm.at[idxv], sem).wait()
            return carry

        lax.fori_loop(0, per_worker, body, 0)

    def fast_path():
        xg = sc_gather(x, idx)
        og = pl.pallas_call(
            _tc_matmul_body,
            grid=(capb,),
            in_specs=[pl.BlockSpec((blk, d), lambda i: (i, 0)),
                      pl.BlockSpec((d, d), lambda i: (0, 0))],
            out_specs=pl.BlockSpec((blk, d), lambda i: (i, 0)),
            out_shape=jax.ShapeDtypeStruct((cap, d), jnp.float32),
        )(xg, mt)
        mlp_out = pl.pallas_call(
            _tc_mlp_body,
            grid=(nblocks,),
            in_specs=[pl.BlockSpec((blk, 1), lambda i: (i, 0)),
                      pl.BlockSpec((blk, d), lambda i: (i, 0))] + w_specs,
            out_specs=pl.BlockSpec((blk, d), lambda i: (i, 0)),
            out_shape=jax.ShapeDtypeStruct((n, d), jnp.float32),
        )(mask2, x, *weights)
        out_ref = jax.new_ref(mlp_out)
        sc_scatter(og, idx, out_ref)
        return jax.freeze(out_ref)

    def slow_path():
        return pl.pallas_call(
            _tc_mono_body,
            grid=(nblocks,),
            in_specs=[pl.BlockSpec((blk, 1), lambda i: (i, 0)),
                      pl.BlockSpec((blk, d), lambda i: (i, 0)),
                      pl.BlockSpec((d, d), lambda i: (0, 0))] + w_specs,
            out_specs=pl.BlockSpec((blk, d), lambda i: (i, 0)),
            out_shape=jax.ShapeDtypeStruct((n, d), jnp.float32),
        )(mask2, x, mt, *weights)

    return lax.cond((n0 > 0) & (n0 <= cap), fast_path, slow_path)
